# baseline (speedup 1.0000x reference)
import functools

import jax
import jax.numpy as jnp
from jax import lax
from jax.experimental import pallas as pl
from jax.experimental.pallas import tpu as pltpu
from jax.experimental.pallas import tpu_sc as plsc

B = 16384
GMF_D = 8
MLP_D = 16
GMF_PACK = 128 // GMF_D
MLP_PACK = 128 // MLP_D
CHUNK = 128
BLK = 1024


def _gather_sc(idx_all, gu_t, gi_t, mu_t, mi_t):
    info = plsc.get_sparse_core_info()
    NW = info.num_cores * info.num_subcores
    BW = B // NW
    n_rounds = BW // CHUNK

    mesh = plsc.VectorSubcoreMesh(core_axis_name="c", subcore_axis_name="s")

    @functools.partial(
        pl.kernel,
        mesh=mesh,
        out_type=[jax.ShapeDtypeStruct((B, 128), jnp.float32)] * 4,
        scratch_types=[
            pltpu.VMEM((4, BW), jnp.int32),
            pltpu.VMEM((CHUNK, 128), jnp.float32),
            pltpu.VMEM((CHUNK, 128), jnp.float32),
            pltpu.VMEM((CHUNK, 128), jnp.float32),
            pltpu.VMEM((CHUNK, 128), jnp.float32),
            pltpu.SemaphoreType.DMA,
        ],
    )
    def gather_kernel(idx_hbm, gu_tab, gi_tab, mu_tab, mi_tab,
                      gu_out, gi_out, mu_out, mi_out,
                      sidx, b0, b1, b2, b3, sem):
        wid = lax.axis_index("s") * info.num_cores + lax.axis_index("c")
        tabs = (gu_tab, gi_tab, mu_tab, mi_tab)
        bufs = (b0, b1, b2, b3)
        outs = (gu_out, gi_out, mu_out, mi_out)
        for t in range(4):
            pltpu.sync_copy(idx_hbm.at[t, wid], sidx.at[t])
        for r in range(n_rounds):
            copies = [
                pltpu.async_copy(
                    tabs[t].at[sidx.at[t, pl.ds(r * CHUNK, CHUNK)]],
                    bufs[t], sem)
                for t in range(4)
            ]
            for c in copies:
                c.wait()
            for t in range(4):
                pltpu.sync_copy(
                    bufs[t], outs[t].at[pl.ds(wid * BW + r * CHUNK, CHUNK)])

    return gather_kernel(idx_all, gu_t, gi_t, mu_t, mi_t)


def _head_tc_body(u, it, gu128, gi128, mu128, mi128,
                  w1u, w1i, b1, w2, b2, wlg, wlh, bl, out):
    liota = lax.broadcasted_iota(jnp.int32, (BLK, 128), 1)
    g_sel = lax.broadcasted_iota(jnp.int32, (128, GMF_D), 0) % GMF_D
    g_pos = lax.broadcasted_iota(jnp.int32, (128, GMF_D), 1)
    G8 = (g_sel == g_pos).astype(jnp.float32)
    m_sel = lax.broadcasted_iota(jnp.int32, (128, MLP_D), 0) % MLP_D
    m_pos = lax.broadcasted_iota(jnp.int32, (128, MLP_D), 1)
    G16 = (m_sel == m_pos).astype(jnp.float32)

    u_g = u[...] % GMF_PACK
    i_g = it[...] % GMF_PACK
    u_m = u[...] % MLP_PACK
    i_m = it[...] % MLP_PACK
    gu = jnp.where(liota // GMF_D == u_g, gu128[...], 0.0) @ G8
    gi = jnp.where(liota // GMF_D == i_g, gi128[...], 0.0) @ G8
    mu = jnp.where(liota // MLP_D == u_m, mu128[...], 0.0) @ G16
    mi = jnp.where(liota // MLP_D == i_m, mi128[...], 0.0) @ G16
    gmf = gu * gi

    h = mu @ w1u[...] + mi @ w1i[...] + b1[...]
    h = jnp.maximum(h, 0.0)
    h = h @ w2[...] + b2[...]
    h = jnp.maximum(h, 0.0)
    logits = gmf @ wlg[...] + h @ wlh[...] + bl[...]
    out[...] = jax.nn.sigmoid(logits)


def kernel(user, item, gmf_user_emb, gmf_item_emb, mlp_user_emb, mlp_item_emb,
           W1, b1, W2, b2, Wl, bl):
    u32 = user.astype(jnp.int32)
    i32 = item.astype(jnp.int32)
    nw = B // 512
    idx_all = jnp.stack([
        (u32 // GMF_PACK).reshape(nw, 512),
        (i32 // GMF_PACK).reshape(nw, 512),
        (u32 // MLP_PACK).reshape(nw, 512),
        (i32 // MLP_PACK).reshape(nw, 512),
    ])
    gu_t = gmf_user_emb.reshape(-1, 128)
    gi_t = gmf_item_emb.reshape(-1, 128)
    mu_t = mlp_user_emb.reshape(-1, 128)
    mi_t = mlp_item_emb.reshape(-1, 128)

    gu128, gi128, mu128, mi128 = _gather_sc(idx_all, gu_t, gi_t, mu_t, mi_t)

    w1u = W1[:MLP_D]
    w1i = W1[MLP_D:]
    wlg = Wl[:GMF_D]
    wlh = Wl[GMF_D:]
    b1r = b1.reshape(1, -1)
    b2r = b2.reshape(1, -1)
    blr = bl.reshape(1, 1)
    u2d = u32.reshape(B, 1)
    i2d = i32.reshape(B, 1)

    n_blk = B // BLK
    out = pl.pallas_call(
        _head_tc_body,
        grid=(n_blk,),
        in_specs=[
            pl.BlockSpec((BLK, 1), lambda i: (i, 0)),
            pl.BlockSpec((BLK, 1), lambda i: (i, 0)),
            pl.BlockSpec((BLK, 128), lambda i: (i, 0)),
            pl.BlockSpec((BLK, 128), lambda i: (i, 0)),
            pl.BlockSpec((BLK, 128), lambda i: (i, 0)),
            pl.BlockSpec((BLK, 128), lambda i: (i, 0)),
            pl.BlockSpec((MLP_D, MLP_D), lambda i: (0, 0)),
            pl.BlockSpec((MLP_D, MLP_D), lambda i: (0, 0)),
            pl.BlockSpec((1, MLP_D), lambda i: (0, 0)),
            pl.BlockSpec((MLP_D, GMF_D), lambda i: (0, 0)),
            pl.BlockSpec((1, GMF_D), lambda i: (0, 0)),
            pl.BlockSpec((GMF_D, 1), lambda i: (0, 0)),
            pl.BlockSpec((GMF_D, 1), lambda i: (0, 0)),
            pl.BlockSpec((1, 1), lambda i: (0, 0)),
        ],
        out_specs=pl.BlockSpec((BLK, 1), lambda i: (i, 0)),
        out_shape=jax.ShapeDtypeStruct((B, 1), jnp.float32),
    )(u2d, i2d, gu128, gi128, mu128, mi128,
      w1u, w1i, b1r, W2, b2r, wlg, wlh, blr)
    return out.reshape(-1)
